# baseline (speedup 1.0000x reference)
import functools

import jax
import jax.numpy as jnp
from jax import lax
from jax.experimental import pallas as pl
from jax.experimental.pallas import tpu as pltpu
from jax.experimental.pallas import tpu_sc as plsc

_D = 768
_BATCH = 4
_SEQ = 2048
_NC = 2
_NS = 16
_NW = _NC * _NS
_PP = _SEQ // _NW
_C = 8
_NSTEP = _PP // _C
_ROWS = _BATCH * _C
_LPT = _D // 16


@functools.partial(
    pl.kernel,
    out_type=jax.ShapeDtypeStruct((_BATCH * _SEQ, _D), jnp.float32),
    mesh=plsc.VectorSubcoreMesh(core_axis_name="c", subcore_axis_name="s"),
    scratch_types=[
        pltpu.VMEM((_BATCH, _PP), jnp.int32),
        [pltpu.VMEM((_ROWS, _D), jnp.float32) for _ in range(3)],
        pltpu.VMEM((_PP, _D), jnp.float32),
        [pltpu.SemaphoreType.DMA for _ in range(3)],
        [pltpu.SemaphoreType.DMA for _ in range(3)],
        pltpu.SemaphoreType.DMA,
        pltpu.SemaphoreType.DMA,
    ],
)
def _emb_kernel(x_hbm, wte_hbm, wpe_hbm, out_hbm,
                idx_v, rows, wpe_v, gsem, ssem, isem, wsem):
    wid = lax.axis_index("s") * _NC + lax.axis_index("c")
    pos0 = wid * _PP

    idx_cp = [
        pltpu.async_copy(x_hbm.at[pl.ds(b * _SEQ + pos0, _PP)],
                         idx_v.at[b], isem)
        for b in range(_BATCH)
    ]
    wpe_cp = pltpu.async_copy(wpe_hbm.at[pl.ds(pos0, _PP)], wpe_v, wsem)
    for cp in idx_cp:
        cp.wait()

    def gather(i):
        buf = rows[i % 3]
        return [
            pltpu.async_copy(
                wte_hbm.at[idx_v.at[b, pl.ds(i * _C, _C)]],
                buf.at[pl.ds(b * _C, _C)], gsem[i % 3])
            for b in range(_BATCH)
        ]

    g_cp = {0: gather(0), 1: gather(1)}
    s_cp = {}
    for i in range(_NSTEP):
        if i == 0:
            wpe_cp.wait()
        for cp in g_cp[i]:
            cp.wait()
        buf = rows[i % 3]

        def tok_body(t, carry):
            for dd in range(_LPT):
                sl = pl.ds(dd * 16, 16)
                wv = wpe_v[i * _C + t, sl]
                for b in range(_BATCH):
                    r = b * _C + t
                    buf[r, sl] = buf[r, sl] + wv
            return carry

        lax.fori_loop(0, _C, tok_body, 0)
        if i + 2 < _NSTEP:
            if i - 1 >= 0:
                for cp in s_cp[i - 1]:
                    cp.wait()
            g_cp[i + 2] = gather(i + 2)
        s_cp[i] = [
            pltpu.async_copy(
                buf.at[pl.ds(b * _C, _C)],
                out_hbm.at[pl.ds(b * _SEQ + pos0 + i * _C, _C)], ssem[i % 3])
            for b in range(_BATCH)
        ]
    for i in range(_NSTEP - 3, _NSTEP):
        for cp in s_cp[i]:
            cp.wait()


def kernel(X, wte, wpe):
    xf = X.reshape(-1).astype(jnp.int32)
    out = _emb_kernel(xf, wte, wpe)
    return out.reshape(_BATCH, _SEQ, _D)


# speedup vs baseline: 1.0278x; 1.0278x over previous
import functools

import jax
import jax.numpy as jnp
from jax import lax
from jax.experimental import pallas as pl
from jax.experimental.pallas import tpu as pltpu
from jax.experimental.pallas import tpu_sc as plsc

_D = 768
_BATCH = 4
_SEQ = 2048
_NC = 2
_NS = 16
_NW = _NC * _NS
_PP = _SEQ // _NW
_C = 32
_NK = _PP // _C
_NSTEP = _NK * _BATCH
_LPT = _D // 16


@functools.partial(
    pl.kernel,
    out_type=jax.ShapeDtypeStruct((_BATCH, _SEQ, _D), jnp.float32),
    mesh=plsc.VectorSubcoreMesh(core_axis_name="c", subcore_axis_name="s"),
    scratch_types=[
        pltpu.VMEM((_NSTEP, _C), jnp.int32),
        [pltpu.VMEM((_C, _D), jnp.float32) for _ in range(3)],
        [pltpu.VMEM((_C, _D), jnp.float32) for _ in range(2)],
        [pltpu.SemaphoreType.DMA for _ in range(3)],
        [pltpu.SemaphoreType.DMA for _ in range(3)],
        pltpu.SemaphoreType.DMA,
        pltpu.SemaphoreType.DMA,
    ],
)
def _emb_kernel(x_hbm, wte_hbm, wpe_hbm, out_hbm,
                idx_v, rows, wpes, gsem, ssem, isem, wsem):
    wid = lax.axis_index("s") * _NC + lax.axis_index("c")
    pos0 = wid * _PP

    def kb(i):
        k, b = divmod(i, _BATCH)
        return k, b

    idx_cp = []
    for i in range(_NSTEP):
        k, b = kb(i)
        idx_cp.append(pltpu.async_copy(
            x_hbm.at[b, pl.ds(pos0 + k * _C, _C)], idx_v.at[i], isem))
    for cp in idx_cp:
        cp.wait()

    pltpu.sync_copy(wpe_hbm.at[pl.ds(pos0, _C)], wpes[0])
    wpe_cp = pltpu.async_copy(wpe_hbm.at[pl.ds(pos0 + _C, _C)], wpes[1], wsem)

    def gather(i):
        return pltpu.async_copy(wte_hbm.at[idx_v.at[i]], rows[i % 3],
                                gsem[i % 3])

    g_cp = {0: gather(0), 1: gather(1)}
    s_cp = {}
    for i in range(_NSTEP):
        k, b = kb(i)
        if i == _BATCH:
            wpe_cp.wait()
        g_cp[i].wait()
        buf, wpe_b = rows[i % 3], wpes[k]

        @plsc.parallel_loop(0, _C)
        def tok_body(t):
            for dd in range(_LPT):
                sl = pl.ds(dd * 16, 16)
                buf[t, sl] = buf[t, sl] + wpe_b[t, sl]

        if i + 2 < _NSTEP:
            if i - 1 >= 0:
                s_cp[i - 1].wait()
            g_cp[i + 2] = gather(i + 2)
        s_cp[i] = pltpu.async_copy(
            buf, out_hbm.at[b, pl.ds(pos0 + k * _C, _C)], ssem[i % 3])
    for i in range(_NSTEP - 3, _NSTEP):
        s_cp[i].wait()


def kernel(X, wte, wpe):
    return _emb_kernel(X.astype(jnp.int32), wte, wpe)


# speedup vs baseline: 1.0671x; 1.0383x over previous
import functools

import jax
import jax.numpy as jnp
from jax import lax
from jax.experimental import pallas as pl
from jax.experimental.pallas import tpu as pltpu
from jax.experimental.pallas import tpu_sc as plsc

_D = 768
_BATCH = 4
_SEQ = 2048
_NC = 2
_NS = 16
_NW = _NC * _NS
_PP = _SEQ // _NW
_C = 32
_NK = _PP // _C
_NSTEP = _NK * _BATCH
_LPT = _D // 16


@functools.partial(
    pl.kernel,
    out_type=jax.ShapeDtypeStruct((_BATCH, _SEQ, _D), jnp.float32),
    mesh=plsc.VectorSubcoreMesh(core_axis_name="c", subcore_axis_name="s"),
    scratch_types=[
        pltpu.VMEM((_NSTEP, _C), jnp.int32),
        [pltpu.VMEM((_C, _D), jnp.float32) for _ in range(3)],
        [pltpu.VMEM((_C, _D), jnp.float32) for _ in range(2)],
        [pltpu.SemaphoreType.DMA for _ in range(3)],
        [pltpu.SemaphoreType.DMA for _ in range(3)],
        pltpu.SemaphoreType.DMA,
        pltpu.SemaphoreType.DMA,
    ],
)
def _emb_kernel(x_hbm, wte_hbm, wpe_hbm, out_hbm,
                idx_v, rows, wpes, gsem, ssem, isem, wsem):
    wid = lax.axis_index("s") * _NC + lax.axis_index("c")
    pos0 = wid * _PP

    def kb(i):
        k, b = divmod(i, _BATCH)
        return k, b

    idx_cp = []
    for i in range(_NSTEP):
        k, b = kb(i)
        idx_cp.append(pltpu.async_copy(
            x_hbm.at[b, pl.ds(pos0 + k * _C, _C)], idx_v.at[i], isem))
    for cp in idx_cp:
        cp.wait()

    pltpu.sync_copy(wpe_hbm.at[pl.ds(pos0, _C)], wpes[0])
    wpe_cp = pltpu.async_copy(wpe_hbm.at[pl.ds(pos0 + _C, _C)], wpes[1], wsem)

    def gather(i):
        return pltpu.async_copy(wte_hbm.at[idx_v.at[i]], rows[i % 3],
                                gsem[i % 3])

    g_cp = {0: gather(0), 1: gather(1)}
    s_cp = {}
    for i in range(_NSTEP):
        k, b = kb(i)
        if i == _BATCH:
            wpe_cp.wait()
        g_cp[i].wait()
        buf, wpe_b = rows[i % 3], wpes[k]

        def tok_body(t, carry):
            for dd in range(_LPT):
                sl = pl.ds(dd * 16, 16)
                buf[t, sl] = buf[t, sl] + wpe_b[t, sl]
            return carry

        lax.fori_loop(0, _C, tok_body, 0)

        if i + 2 < _NSTEP:
            if i - 1 >= 0:
                s_cp[i - 1].wait()
            g_cp[i + 2] = gather(i + 2)
        s_cp[i] = pltpu.async_copy(
            buf, out_hbm.at[b, pl.ds(pos0 + k * _C, _C)], ssem[i % 3])
    for i in range(_NSTEP - 3, _NSTEP):
        s_cp[i].wait()


def kernel(X, wte, wpe):
    return _emb_kernel(X.astype(jnp.int32), wte, wpe)
